# baseline (speedup 1.0000x reference)
import jax
import jax.numpy as jnp
from jax import lax
from jax.experimental import pallas as pl
from jax.experimental.pallas import tpu as pltpu
from jax.experimental.pallas import tpu_sc as plsc

N = 10000
E = 320000
C0 = 128
H = 256
NC = 2
NS = 16
KB = 80
EP = 322560
NP = 10240
RPT = NP // NS


def _zero_acc(z128_hbm, acc, r0):
  pltpu.sync_copy(z128_hbm.at[pl.ds(r0, RPT)], acc.at[pl.ds(r0, RPT)])


def _edge_pipeline(nb, ebase, src_hbm, dst_hbm, w_hbm, tab_hbm, bias, acc,
                   z128_hbm, bufs_a, bufs_b):
  def issue_idx(bufs, b):
    base = ebase + b * KB
    pltpu.async_copy(src_hbm.at[pl.ds(base, KB)], bufs[0], bufs[8])
    pltpu.async_copy(dst_hbm.at[pl.ds(base, KB)], bufs[1], bufs[8])
    pltpu.async_copy(w_hbm.at[pl.ds(base, KB)], bufs[2], bufs[8])

  def drain_idx(bufs):
    pltpu.make_async_copy(src_hbm.at[pl.ds(0, KB)], bufs[0], bufs[8]).wait()
    pltpu.make_async_copy(dst_hbm.at[pl.ds(0, KB)], bufs[1], bufs[8]).wait()
    pltpu.make_async_copy(w_hbm.at[pl.ds(0, KB)], bufs[2], bufs[8]).wait()

  def issue_gather(bufs):
    if bias is not None:
      for j in range(KB // 16):
        sl = pl.ds(j * 16, 16)
        bufs[0][sl] = bufs[0][sl] + bias
    pltpu.async_copy(tab_hbm.at[bufs[0]], bufs[3], bufs[6])

  def drain_gather(bufs):
    pltpu.make_async_copy(tab_hbm.at[pl.ds(0, KB)], bufs[3], bufs[6]).wait()

  def scale_into(bufs):
    src_v, dst_v, w_v, rows_v, sdata, sdst = bufs[:6]
    for j in range(KB // 16):
      sl = pl.ds(j * 16, 16)
      sdst[sl] = dst_v[sl]

    def edge(e, carry):
      wspl = plsc.load_gather(w_v, (jnp.full((16,), e, jnp.int32),))
      for j in range(C0 // 16):
        sl = pl.ds(j * 16, 16)
        sdata[e, sl] = rows_v[e, sl] * wspl
      return carry

    lax.fori_loop(0, KB, edge, 0)

  def issue_scatter(bufs):
    pltpu.async_copy(bufs[4], acc.at[bufs[5]], bufs[7], add=True)

  def drain_scatter(bufs):
    pltpu.make_async_copy(z128_hbm.at[pl.ds(0, KB)], bufs[4], bufs[7]).wait()

  for bufs, b0 in ((bufs_a, 0), (bufs_b, 1)):
    issue_idx(bufs, b0)
    drain_idx(bufs)
    issue_gather(bufs)

  def it(g, carry):
    for bufs, off in ((bufs_a, 0), (bufs_b, 1)):
      b = 2 * g + off
      drain_gather(bufs)

      @pl.when(g > 0)
      def _():
        drain_scatter(bufs)

      scale_into(bufs)
      issue_scatter(bufs)

      @pl.when(b + 2 < nb)
      def _():
        issue_idx(bufs, b + 2)

    for bufs, off in ((bufs_a, 0), (bufs_b, 1)):
      @pl.when(2 * g + off + 2 < nb)
      def _():
        drain_idx(bufs)
        issue_gather(bufs)
    return carry

  lax.fori_loop(0, nb // 2, it, 0)
  drain_scatter(bufs_a)
  drain_scatter(bufs_b)


def _sc_layer0(x_hbm, src_hbm, dst_hbm, w_hbm, z128_hbm,
               agg_out, deg_out,
               srcA, dstA, wA, rowsA, sdataA, sdstA,
               srcB, dstB, wB, rowsB, sdataB, sdstB,
               acc,
               gsemA, ssemA, isemA, gsemB, ssemB, isemB):
  c = lax.axis_index("c")
  s = lax.axis_index("s")
  r0 = s * RPT
  _zero_acc(z128_hbm, acc, r0)
  plsc.subcore_barrier()

  epw = EP // (NC * NS)
  ebase = (s * NC + c) * epw
  nb = epw // KB

  bufs_a = (srcA, dstA, wA, rowsA, sdataA, sdstA, gsemA, ssemA, isemA)
  bufs_b = (srcB, dstB, wB, rowsB, sdataB, sdstB, gsemB, ssemB, isemB)
  _edge_pipeline(nb, ebase, src_hbm, dst_hbm, w_hbm, x_hbm, None, acc,
                 z128_hbm, bufs_a, bufs_b)
  plsc.subcore_barrier()
  pltpu.sync_copy(acc.at[pl.ds(r0, RPT)], agg_out.at[c, pl.ds(r0, RPT)])
  plsc.subcore_barrier()

  _zero_acc(z128_hbm, acc, r0)

  def fill_ones(e, carry):
    for j in range(C0 // 16):
      rowsA[e, pl.ds(j * 16, 16)] = jnp.full((16,), 1.0, jnp.float32)
    return carry

  lax.fori_loop(0, KB, fill_ones, 0)
  plsc.subcore_barrier()

  def dit(g, carry):
    for sdst, ssem, sdata, off in ((sdstA, ssemA, sdataA, 0),
                                   (sdstB, ssemB, sdataB, 1)):
      b = 2 * g + off

      @pl.when(g > 0)
      def _():
        pltpu.make_async_copy(z128_hbm.at[pl.ds(0, KB)], sdata, ssem).wait()

      base = ebase + b * KB
      pltpu.sync_copy(dst_hbm.at[pl.ds(base, KB)], sdst)
      pltpu.async_copy(rowsA, acc.at[sdst], ssem, add=True)
    return carry

  lax.fori_loop(0, nb // 2, dit, 0)
  pltpu.make_async_copy(z128_hbm.at[pl.ds(0, KB)], sdataA, ssemA).wait()
  pltpu.make_async_copy(z128_hbm.at[pl.ds(0, KB)], sdataB, ssemB).wait()
  plsc.subcore_barrier()
  pltpu.sync_copy(acc.at[pl.ds(r0, RPT)], deg_out.at[c, pl.ds(r0, RPT)])


def _sc_layer_k(tab_hbm, src_hbm, dst_hbm, w_hbm, z128_hbm,
                agg_out,
                srcA, dstA, wA, rowsA, sdataA, sdstA,
                srcB, dstB, wB, rowsB, sdataB, sdstB,
                acc,
                gsemA, ssemA, isemA, gsemB, ssemB, isemB):
  c = lax.axis_index("c")
  s = lax.axis_index("s")
  r0 = s * RPT
  _zero_acc(z128_hbm, acc, r0)
  plsc.subcore_barrier()

  epw = EP // NS
  ebase = s * epw
  nb = epw // KB

  bufs_a = (srcA, dstA, wA, rowsA, sdataA, sdstA, gsemA, ssemA, isemA)
  bufs_b = (srcB, dstB, wB, rowsB, sdataB, sdstB, gsemB, ssemB, isemB)
  _edge_pipeline(nb, ebase, src_hbm, dst_hbm, w_hbm, tab_hbm, c * N, acc,
                 z128_hbm, bufs_a, bufs_b)
  plsc.subcore_barrier()
  pltpu.sync_copy(acc.at[pl.ds(r0, RPT)], agg_out.at[c, pl.ds(r0, RPT)])


_SC_MESH = plsc.VectorSubcoreMesh(core_axis_name="c", subcore_axis_name="s")
_SC_PARAMS = pltpu.CompilerParams(needs_layout_passes=False)

_SC_SCRATCH = [
    pltpu.VMEM((KB,), jnp.int32),
    pltpu.VMEM((KB,), jnp.int32),
    pltpu.VMEM((KB,), jnp.float32),
    pltpu.VMEM((KB, C0), jnp.float32),
    pltpu.VMEM((KB, C0), jnp.float32),
    pltpu.VMEM((KB,), jnp.int32),
    pltpu.VMEM((KB,), jnp.int32),
    pltpu.VMEM((KB,), jnp.int32),
    pltpu.VMEM((KB,), jnp.float32),
    pltpu.VMEM((KB, C0), jnp.float32),
    pltpu.VMEM((KB, C0), jnp.float32),
    pltpu.VMEM((KB,), jnp.int32),
    pltpu.VMEM_SHARED((NP, C0), jnp.float32),
    pltpu.SemaphoreType.DMA,
    pltpu.SemaphoreType.DMA,
    pltpu.SemaphoreType.DMA,
    pltpu.SemaphoreType.DMA,
    pltpu.SemaphoreType.DMA,
    pltpu.SemaphoreType.DMA,
]

_sc_layer0_call = pl.kernel(
    _sc_layer0,
    out_type=(
        jax.ShapeDtypeStruct((NC, NP, C0), jnp.float32),
        jax.ShapeDtypeStruct((NC, NP, C0), jnp.float32),
    ),
    mesh=_SC_MESH,
    scratch_types=list(_SC_SCRATCH),
    compiler_params=_SC_PARAMS,
)

_sc_layer_k_call = pl.kernel(
    _sc_layer_k,
    out_type=jax.ShapeDtypeStruct((NC, NP, C0), jnp.float32),
    mesh=_SC_MESH,
    scratch_types=list(_SC_SCRATCH),
    compiler_params=_SC_PARAMS,
)

_R = 1000


def _recip_deg(deg_ref):
  deg = deg_ref[0, :, 0:1] + deg_ref[1, :, 0:1]
  return 1.0 / jnp.maximum(deg, 1.0)


def _tc_layer0(eps_ref, x_ref, agg_ref, deg_ref, w_ref, b_ref, out_ref):
  recip = _recip_deg(deg_ref)
  neigh = (agg_ref[0] + agg_ref[1]) * recip
  rst = (1.0 + eps_ref[0]) * x_ref[...] + neigh
  h = lax.dot_general(rst, w_ref[...], (((1,), (1,)), ((), ())),
                      preferred_element_type=jnp.float32) + b_ref[...]
  h = jnp.maximum(h, 0.0)
  out_ref[0] = h[:, :C0]
  out_ref[1] = h[:, C0:]


def _tc_layer1(eps_ref, hh_ref, agg_ref, deg_ref, w_ref, b_ref, out_ref):
  recip = _recip_deg(deg_ref)
  hcat = jnp.concatenate([hh_ref[0], hh_ref[1]], axis=1)
  neigh = jnp.concatenate([agg_ref[0], agg_ref[1]], axis=1) * recip
  rst = (1.0 + eps_ref[1]) * hcat + neigh
  h = lax.dot_general(rst, w_ref[...], (((1,), (1,)), ((), ())),
                      preferred_element_type=jnp.float32) + b_ref[...]
  h = jnp.maximum(h, 0.0)
  out_ref[0] = h[:, :C0]
  out_ref[1] = h[:, C0:]


def _tc_layer2(eps_ref, hh_ref, agg_ref, deg_ref, w_ref, b_ref, out_ref):
  recip = _recip_deg(deg_ref)
  hcat = jnp.concatenate([hh_ref[0], hh_ref[1]], axis=1)
  neigh = jnp.concatenate([agg_ref[0], agg_ref[1]], axis=1) * recip
  rst = (1.0 + eps_ref[2]) * hcat + neigh
  h = lax.dot_general(rst, w_ref[...], (((1,), (1,)), ((), ())),
                      preferred_element_type=jnp.float32) + b_ref[...]
  out_ref[...] = h

  @pl.when(pl.program_id(0) == 0)
  def _():
    out_ref[0:1, :] = jnp.zeros((1, H), jnp.float32)


def _tc_call(body, in_specs, out_specs, out_shape):
  return pl.pallas_call(
      body,
      grid=(N // _R,),
      in_specs=in_specs,
      out_specs=out_specs,
      out_shape=out_shape,
  )


_eps_spec = pl.BlockSpec(memory_space=pltpu.SMEM)
_w1_spec = pl.BlockSpec((H, C0), lambda i: (0, 0))
_w2_spec = pl.BlockSpec((H, H), lambda i: (0, 0))
_b_spec = pl.BlockSpec((1, H), lambda i: (0, 0))
_half_spec = pl.BlockSpec((NC, _R, C0), lambda i: (0, i, 0))
_x_spec = pl.BlockSpec((_R, C0), lambda i: (i, 0))
_full_spec = pl.BlockSpec((_R, H), lambda i: (i, 0))

_tc0_call = _tc_call(
    _tc_layer0,
    [_eps_spec, _x_spec, _half_spec, _half_spec, _w1_spec, _b_spec],
    _half_spec,
    jax.ShapeDtypeStruct((NC, N, C0), jnp.float32),
)

_tc1_call = _tc_call(
    _tc_layer1,
    [_eps_spec, _half_spec, _half_spec, _half_spec, _w2_spec, _b_spec],
    _half_spec,
    jax.ShapeDtypeStruct((NC, N, C0), jnp.float32),
)

_tc2_call = _tc_call(
    _tc_layer2,
    [_eps_spec, _half_spec, _half_spec, _half_spec, _w2_spec, _b_spec],
    _full_spec,
    jax.ShapeDtypeStruct((N, H), jnp.float32),
)


@jax.jit
def kernel(x, edge_index, edge_w, W1, b1, W2, b2, eps):
  pad = EP - E
  src = jnp.concatenate([edge_index[0], jnp.zeros((pad,), jnp.int32)])
  dst = jnp.concatenate([edge_index[1], jnp.full((pad,), N, jnp.int32)])
  edge_w = jnp.concatenate([edge_w, jnp.zeros((pad,), jnp.float32)])
  z128 = jnp.zeros((NP, C0), jnp.float32)
  b1r = b1.reshape(1, H)
  b2r = b2.reshape(1, H)

  agg0, dpart = _sc_layer0_call(x, src, dst, edge_w, z128)
  h1 = _tc0_call(eps, x, agg0, dpart, W1, b1r)
  agg1 = _sc_layer_k_call(h1.reshape(NC * N, C0), src, dst, edge_w, z128)
  h2 = _tc1_call(eps, h1, agg1, dpart, W2, b2r)
  agg2 = _sc_layer_k_call(h2.reshape(NC * N, C0), src, dst, edge_w, z128)
  out = _tc2_call(eps, h2, agg2, dpart, W2, b2r)
  return out


# speedup vs baseline: 1.1964x; 1.1964x over previous
import jax
import jax.numpy as jnp
from jax import lax
from jax.experimental import pallas as pl
from jax.experimental.pallas import tpu as pltpu
from jax.experimental.pallas import tpu_sc as plsc

N = 10000
E = 320000
C0 = 128
H = 256
NC = 2
NS = 16
KB = 80
NP = 10240
RPT = NP // NS


def _scale_rows(rows_v, pk_v):
  def edge(e, carry):
    wbits = plsc.load_gather(pk_v, (jnp.full((16,), 4 * e + 3, jnp.int32),))
    wspl = plsc.bitcast(wbits, jnp.float32)
    for j in range(C0 // 16):
      sl = pl.ds(j * 16, 16)
      rows_v[e, sl] = rows_v[e, sl] * wspl
    return carry

  lax.fori_loop(0, KB, edge, 0)


def _unpack_idx(pk_v, col, out_v):
  lanes4 = lax.iota(jnp.int32, 16) * 4
  for g in range(KB // 16):
    idx = lanes4 + (4 * 16 * g) + col
    out_v[pl.ds(g * 16, 16)] = plsc.load_gather(pk_v, (idx,))


def _zero_acc(z128_hbm, acc, r0):
  pltpu.sync_copy(z128_hbm.at[pl.ds(r0, RPT)], acc.at[pl.ds(r0, RPT)])


def _sc_layer0(x_hbm, pk_hbm, z128_hbm,
               agg_out, deg_out,
               src_v, dst_v, pk_v, rows_v, ones_v, acc, sem):
  c = lax.axis_index("c")
  s = lax.axis_index("s")
  r0 = s * RPT
  _zero_acc(z128_hbm, acc, r0)

  def fill_ones(e, carry):
    for j in range(C0 // 16):
      ones_v[e, pl.ds(j * 16, 16)] = jnp.full((16,), 1.0, jnp.float32)
    return carry

  lax.fori_loop(0, KB, fill_ones, 0)
  plsc.subcore_barrier()

  epw = E // (NC * NS)
  base0 = (s * NC + c) * epw
  nb = epw // KB

  def batch(i, carry):
    base = base0 + i * KB
    pltpu.sync_copy(pk_hbm.at[pl.ds(4 * base, 4 * KB)], pk_v)
    _unpack_idx(pk_v, 0, src_v)
    _unpack_idx(pk_v, 2, dst_v)
    pltpu.async_copy(x_hbm.at[src_v], rows_v, sem).wait()
    _scale_rows(rows_v, pk_v)
    pltpu.sync_copy(rows_v, acc.at[dst_v], add=True)
    return carry

  lax.fori_loop(0, nb, batch, 0)
  plsc.subcore_barrier()
  pltpu.sync_copy(acc.at[pl.ds(r0, RPT)], agg_out.at[c, pl.ds(r0, RPT)])
  plsc.subcore_barrier()

  _zero_acc(z128_hbm, acc, r0)
  plsc.subcore_barrier()

  def dbatch(i, carry):
    base = base0 + i * KB
    pltpu.sync_copy(pk_hbm.at[pl.ds(4 * base, 4 * KB)], pk_v)
    _unpack_idx(pk_v, 2, dst_v)
    pltpu.sync_copy(ones_v, acc.at[dst_v], add=True)
    return carry

  lax.fori_loop(0, nb, dbatch, 0)
  plsc.subcore_barrier()
  pltpu.sync_copy(acc.at[pl.ds(r0, RPT)], deg_out.at[c, pl.ds(r0, RPT)])


def _sc_layer_k(tab_hbm, pk_hbm, z128_hbm,
                agg_out,
                src_v, dst_v, pk_v, rows_v, acc, sem):
  c = lax.axis_index("c")
  s = lax.axis_index("s")
  r0 = s * RPT
  _zero_acc(z128_hbm, acc, r0)
  plsc.subcore_barrier()

  epw = E // NS
  base0 = s * epw
  nb = epw // KB

  def batch(i, carry):
    base = base0 + i * KB
    pltpu.sync_copy(pk_hbm.at[pl.ds(4 * base, 4 * KB)], pk_v)
    _unpack_idx(pk_v, c, src_v)
    _unpack_idx(pk_v, 2, dst_v)
    pltpu.async_copy(tab_hbm.at[src_v], rows_v, sem).wait()
    _scale_rows(rows_v, pk_v)
    pltpu.sync_copy(rows_v, acc.at[dst_v], add=True)
    return carry

  lax.fori_loop(0, nb, batch, 0)
  plsc.subcore_barrier()
  pltpu.sync_copy(acc.at[pl.ds(r0, RPT)], agg_out.at[c, pl.ds(r0, RPT)])


_SC_MESH = plsc.VectorSubcoreMesh(core_axis_name="c", subcore_axis_name="s")
_SC_PARAMS = pltpu.CompilerParams(needs_layout_passes=False)

_sc_layer0_call = pl.kernel(
    _sc_layer0,
    out_type=(
        jax.ShapeDtypeStruct((NC, NP, C0), jnp.float32),
        jax.ShapeDtypeStruct((NC, NP, C0), jnp.float32),
    ),
    mesh=_SC_MESH,
    scratch_types=[
        pltpu.VMEM((KB,), jnp.int32),
        pltpu.VMEM((KB,), jnp.int32),
        pltpu.VMEM((4 * KB,), jnp.int32),
        pltpu.VMEM((KB, C0), jnp.float32),
        pltpu.VMEM((KB, C0), jnp.float32),
        pltpu.VMEM_SHARED((NP, C0), jnp.float32),
        pltpu.SemaphoreType.DMA,
    ],
    compiler_params=_SC_PARAMS,
)

_sc_layer_k_call = pl.kernel(
    _sc_layer_k,
    out_type=jax.ShapeDtypeStruct((NC, NP, C0), jnp.float32),
    mesh=_SC_MESH,
    scratch_types=[
        pltpu.VMEM((KB,), jnp.int32),
        pltpu.VMEM((KB,), jnp.int32),
        pltpu.VMEM((4 * KB,), jnp.int32),
        pltpu.VMEM((KB, C0), jnp.float32),
        pltpu.VMEM_SHARED((NP, C0), jnp.float32),
        pltpu.SemaphoreType.DMA,
    ],
    compiler_params=_SC_PARAMS,
)

_R = 1000


def _recip_deg(deg_ref):
  deg = deg_ref[0, :, 0:1] + deg_ref[1, :, 0:1]
  return 1.0 / jnp.maximum(deg, 1.0)


def _tc_layer0(eps_ref, x_ref, agg_ref, deg_ref, w_ref, b_ref, out_ref):
  recip = _recip_deg(deg_ref)
  neigh = (agg_ref[0] + agg_ref[1]) * recip
  rst = (1.0 + eps_ref[0]) * x_ref[...] + neigh
  h = lax.dot_general(rst, w_ref[...], (((1,), (1,)), ((), ())),
                      preferred_element_type=jnp.float32) + b_ref[...]
  h = jnp.maximum(h, 0.0)
  out_ref[0] = h[:, :C0]
  out_ref[1] = h[:, C0:]


def _tc_layer1(eps_ref, hh_ref, agg_ref, deg_ref, w_ref, b_ref, out_ref):
  recip = _recip_deg(deg_ref)
  hcat = jnp.concatenate([hh_ref[0], hh_ref[1]], axis=1)
  neigh = jnp.concatenate([agg_ref[0], agg_ref[1]], axis=1) * recip
  rst = (1.0 + eps_ref[1]) * hcat + neigh
  h = lax.dot_general(rst, w_ref[...], (((1,), (1,)), ((), ())),
                      preferred_element_type=jnp.float32) + b_ref[...]
  h = jnp.maximum(h, 0.0)
  out_ref[0] = h[:, :C0]
  out_ref[1] = h[:, C0:]


def _tc_layer2(eps_ref, hh_ref, agg_ref, deg_ref, w_ref, b_ref, out_ref):
  recip = _recip_deg(deg_ref)
  hcat = jnp.concatenate([hh_ref[0], hh_ref[1]], axis=1)
  neigh = jnp.concatenate([agg_ref[0], agg_ref[1]], axis=1) * recip
  rst = (1.0 + eps_ref[2]) * hcat + neigh
  h = lax.dot_general(rst, w_ref[...], (((1,), (1,)), ((), ())),
                      preferred_element_type=jnp.float32) + b_ref[...]
  out_ref[...] = h

  @pl.when(pl.program_id(0) == 0)
  def _():
    out_ref[0:1, :] = jnp.zeros((1, H), jnp.float32)


def _tc_call(body, in_specs, out_specs, out_shape):
  return pl.pallas_call(
      body,
      grid=(N // _R,),
      in_specs=in_specs,
      out_specs=out_specs,
      out_shape=out_shape,
  )


_eps_spec = pl.BlockSpec(memory_space=pltpu.SMEM)
_w1_spec = pl.BlockSpec((H, C0), lambda i: (0, 0))
_w2_spec = pl.BlockSpec((H, H), lambda i: (0, 0))
_b_spec = pl.BlockSpec((1, H), lambda i: (0, 0))
_half_spec = pl.BlockSpec((NC, _R, C0), lambda i: (0, i, 0))
_x_spec = pl.BlockSpec((_R, C0), lambda i: (i, 0))
_full_spec = pl.BlockSpec((_R, H), lambda i: (i, 0))

_tc0_call = _tc_call(
    _tc_layer0,
    [_eps_spec, _x_spec, _half_spec, _half_spec, _w1_spec, _b_spec],
    _half_spec,
    jax.ShapeDtypeStruct((NC, N, C0), jnp.float32),
)

_tc1_call = _tc_call(
    _tc_layer1,
    [_eps_spec, _half_spec, _half_spec, _half_spec, _w2_spec, _b_spec],
    _half_spec,
    jax.ShapeDtypeStruct((NC, N, C0), jnp.float32),
)

_tc2_call = _tc_call(
    _tc_layer2,
    [_eps_spec, _half_spec, _half_spec, _half_spec, _w2_spec, _b_spec],
    _full_spec,
    jax.ShapeDtypeStruct((N, H), jnp.float32),
)


@jax.jit
def kernel(x, edge_index, edge_w, W1, b1, W2, b2, eps):
  src = edge_index[0]
  dst = edge_index[1]
  wbits = lax.bitcast_convert_type(edge_w, jnp.int32)
  packed = jnp.stack([src, src + N, dst, wbits], axis=1).reshape(4 * E)
  z128 = jnp.zeros((NP, C0), jnp.float32)
  b1r = b1.reshape(1, H)
  b2r = b2.reshape(1, H)

  agg0, dpart = _sc_layer0_call(x, packed, z128)
  h1 = _tc0_call(eps, x, agg0, dpart, W1, b1r)
  agg1 = _sc_layer_k_call(h1.reshape(NC * N, C0), packed, z128)
  h2 = _tc1_call(eps, h1, agg1, dpart, W2, b2r)
  agg2 = _sc_layer_k_call(h2.reshape(NC * N, C0), packed, z128)
  out = _tc2_call(eps, h2, agg2, dpart, W2, b2r)
  return out


# speedup vs baseline: 1.3552x; 1.1328x over previous
import jax
import jax.numpy as jnp
from jax import lax
from jax.experimental import pallas as pl
from jax.experimental.pallas import tpu as pltpu
from jax.experimental.pallas import tpu_sc as plsc

N = 10000
E = 320000
C0 = 128
H = 256
NC = 2
NS = 16
KB = 80
NP = 10240
RPT = NP // NS


def _scale_rows(rows_v, pk_v):
  def edge(e, carry):
    wbits = plsc.load_gather(pk_v, (jnp.full((16,), 4 * e + 3, jnp.int32),))
    wspl = plsc.bitcast(wbits, jnp.float32)
    for j in range(C0 // 16):
      sl = pl.ds(j * 16, 16)
      rows_v[e, sl] = rows_v[e, sl] * wspl
    return carry

  lax.fori_loop(0, KB, edge, 0)


def _unpack_idx(pk_v, col, out_v):
  lanes4 = lax.iota(jnp.int32, 16) * 4
  for g in range(KB // 16):
    idx = lanes4 + (4 * 16 * g) + col
    out_v[pl.ds(g * 16, 16)] = plsc.load_gather(pk_v, (idx,))


def _edge_loop(nb, base0, tab_hbm, pk_hbm, col, acc, z128_hbm,
               src_v, pk_v, dstA, dstB, rowsA, rowsB, gsem, ssemA, ssemB):
  def half(b, g, dst_v, rows_v, ssem):
    base = base0 + b * KB
    pltpu.sync_copy(pk_hbm.at[pl.ds(4 * base, 4 * KB)], pk_v)

    @pl.when(g > 0)
    def _():
      pltpu.make_async_copy(z128_hbm.at[pl.ds(0, KB)], rows_v, ssem).wait()

    _unpack_idx(pk_v, col, src_v)
    _unpack_idx(pk_v, 2, dst_v)
    pltpu.async_copy(tab_hbm.at[src_v], rows_v, gsem).wait()
    _scale_rows(rows_v, pk_v)
    pltpu.async_copy(rows_v, acc.at[dst_v], ssem, add=True)

  def it(g, carry):
    half(2 * g, g, dstA, rowsA, ssemA)
    half(2 * g + 1, g, dstB, rowsB, ssemB)
    return carry

  lax.fori_loop(0, nb // 2, it, 0)
  pltpu.make_async_copy(z128_hbm.at[pl.ds(0, KB)], rowsA, ssemA).wait()
  pltpu.make_async_copy(z128_hbm.at[pl.ds(0, KB)], rowsB, ssemB).wait()


def _zero_acc(z128_hbm, acc, r0):
  pltpu.sync_copy(z128_hbm.at[pl.ds(r0, RPT)], acc.at[pl.ds(r0, RPT)])


def _sc_layer0(x_hbm, pk_hbm, z128_hbm,
               agg_out, deg_out,
               src_v, dstA, dstB, pk_v, rowsA, rowsB, ones_v, acc,
               gsem, ssemA, ssemB):
  c = lax.axis_index("c")
  s = lax.axis_index("s")
  r0 = s * RPT
  _zero_acc(z128_hbm, acc, r0)

  def fill_ones(e, carry):
    for j in range(C0 // 16):
      ones_v[e, pl.ds(j * 16, 16)] = jnp.full((16,), 1.0, jnp.float32)
    return carry

  lax.fori_loop(0, KB, fill_ones, 0)
  plsc.subcore_barrier()

  epw = E // (NC * NS)
  base0 = (s * NC + c) * epw
  nb = epw // KB

  _edge_loop(nb, base0, x_hbm, pk_hbm, 0, acc, z128_hbm,
             src_v, pk_v, dstA, dstB, rowsA, rowsB, gsem, ssemA, ssemB)
  plsc.subcore_barrier()
  pltpu.sync_copy(acc.at[pl.ds(r0, RPT)], agg_out.at[c, pl.ds(r0, RPT)])
  plsc.subcore_barrier()

  _zero_acc(z128_hbm, acc, r0)
  plsc.subcore_barrier()

  def dbatch(i, carry):
    base = base0 + i * KB
    pltpu.sync_copy(pk_hbm.at[pl.ds(4 * base, 4 * KB)], pk_v)
    _unpack_idx(pk_v, 2, dstA)
    pltpu.sync_copy(ones_v, acc.at[dstA], add=True)
    return carry

  lax.fori_loop(0, nb, dbatch, 0)
  plsc.subcore_barrier()
  pltpu.sync_copy(acc.at[pl.ds(r0, RPT)], deg_out.at[c, pl.ds(r0, RPT)])


def _sc_layer_k(tab_hbm, pk_hbm, z128_hbm,
                agg_out,
                src_v, dstA, dstB, pk_v, rowsA, rowsB, acc,
                gsem, ssemA, ssemB):
  c = lax.axis_index("c")
  s = lax.axis_index("s")
  r0 = s * RPT
  _zero_acc(z128_hbm, acc, r0)
  plsc.subcore_barrier()

  epw = E // NS
  base0 = s * epw
  nb = epw // KB

  _edge_loop(nb, base0, tab_hbm, pk_hbm, c, acc, z128_hbm,
             src_v, pk_v, dstA, dstB, rowsA, rowsB, gsem, ssemA, ssemB)
  plsc.subcore_barrier()
  pltpu.sync_copy(acc.at[pl.ds(r0, RPT)], agg_out.at[c, pl.ds(r0, RPT)])


_SC_MESH = plsc.VectorSubcoreMesh(core_axis_name="c", subcore_axis_name="s")
_SC_PARAMS = pltpu.CompilerParams(needs_layout_passes=False)

_sc_layer0_call = pl.kernel(
    _sc_layer0,
    out_type=(
        jax.ShapeDtypeStruct((NC, NP, C0), jnp.float32),
        jax.ShapeDtypeStruct((NC, NP, C0), jnp.float32),
    ),
    mesh=_SC_MESH,
    scratch_types=[
        pltpu.VMEM((KB,), jnp.int32),
        pltpu.VMEM((KB,), jnp.int32),
        pltpu.VMEM((KB,), jnp.int32),
        pltpu.VMEM((4 * KB,), jnp.int32),
        pltpu.VMEM((KB, C0), jnp.float32),
        pltpu.VMEM((KB, C0), jnp.float32),
        pltpu.VMEM((KB, C0), jnp.float32),
        pltpu.VMEM_SHARED((NP, C0), jnp.float32),
        pltpu.SemaphoreType.DMA,
        pltpu.SemaphoreType.DMA,
        pltpu.SemaphoreType.DMA,
    ],
    compiler_params=_SC_PARAMS,
)

_sc_layer_k_call = pl.kernel(
    _sc_layer_k,
    out_type=jax.ShapeDtypeStruct((NC, NP, C0), jnp.float32),
    mesh=_SC_MESH,
    scratch_types=[
        pltpu.VMEM((KB,), jnp.int32),
        pltpu.VMEM((KB,), jnp.int32),
        pltpu.VMEM((KB,), jnp.int32),
        pltpu.VMEM((4 * KB,), jnp.int32),
        pltpu.VMEM((KB, C0), jnp.float32),
        pltpu.VMEM((KB, C0), jnp.float32),
        pltpu.VMEM_SHARED((NP, C0), jnp.float32),
        pltpu.SemaphoreType.DMA,
        pltpu.SemaphoreType.DMA,
        pltpu.SemaphoreType.DMA,
    ],
    compiler_params=_SC_PARAMS,
)

_R = 1000


def _recip_deg(deg_ref):
  deg = deg_ref[0, :, 0:1] + deg_ref[1, :, 0:1]
  return 1.0 / jnp.maximum(deg, 1.0)


def _tc_layer0(eps_ref, x_ref, agg_ref, deg_ref, w_ref, b_ref, out_ref):
  recip = _recip_deg(deg_ref)
  neigh = (agg_ref[0] + agg_ref[1]) * recip
  rst = (1.0 + eps_ref[0]) * x_ref[...] + neigh
  h = lax.dot_general(rst, w_ref[...], (((1,), (1,)), ((), ())),
                      preferred_element_type=jnp.float32) + b_ref[...]
  h = jnp.maximum(h, 0.0)
  out_ref[0] = h[:, :C0]
  out_ref[1] = h[:, C0:]


def _tc_layer1(eps_ref, hh_ref, agg_ref, deg_ref, w_ref, b_ref, out_ref):
  recip = _recip_deg(deg_ref)
  hcat = jnp.concatenate([hh_ref[0], hh_ref[1]], axis=1)
  neigh = jnp.concatenate([agg_ref[0], agg_ref[1]], axis=1) * recip
  rst = (1.0 + eps_ref[1]) * hcat + neigh
  h = lax.dot_general(rst, w_ref[...], (((1,), (1,)), ((), ())),
                      preferred_element_type=jnp.float32) + b_ref[...]
  h = jnp.maximum(h, 0.0)
  out_ref[0] = h[:, :C0]
  out_ref[1] = h[:, C0:]


def _tc_layer2(eps_ref, hh_ref, agg_ref, deg_ref, w_ref, b_ref, out_ref):
  recip = _recip_deg(deg_ref)
  hcat = jnp.concatenate([hh_ref[0], hh_ref[1]], axis=1)
  neigh = jnp.concatenate([agg_ref[0], agg_ref[1]], axis=1) * recip
  rst = (1.0 + eps_ref[2]) * hcat + neigh
  h = lax.dot_general(rst, w_ref[...], (((1,), (1,)), ((), ())),
                      preferred_element_type=jnp.float32) + b_ref[...]
  out_ref[...] = h

  @pl.when(pl.program_id(0) == 0)
  def _():
    out_ref[0:1, :] = jnp.zeros((1, H), jnp.float32)


def _tc_call(body, in_specs, out_specs, out_shape):
  return pl.pallas_call(
      body,
      grid=(N // _R,),
      in_specs=in_specs,
      out_specs=out_specs,
      out_shape=out_shape,
  )


_eps_spec = pl.BlockSpec(memory_space=pltpu.SMEM)
_w1_spec = pl.BlockSpec((H, C0), lambda i: (0, 0))
_w2_spec = pl.BlockSpec((H, H), lambda i: (0, 0))
_b_spec = pl.BlockSpec((1, H), lambda i: (0, 0))
_half_spec = pl.BlockSpec((NC, _R, C0), lambda i: (0, i, 0))
_x_spec = pl.BlockSpec((_R, C0), lambda i: (i, 0))
_full_spec = pl.BlockSpec((_R, H), lambda i: (i, 0))

_tc0_call = _tc_call(
    _tc_layer0,
    [_eps_spec, _x_spec, _half_spec, _half_spec, _w1_spec, _b_spec],
    _half_spec,
    jax.ShapeDtypeStruct((NC, N, C0), jnp.float32),
)

_tc1_call = _tc_call(
    _tc_layer1,
    [_eps_spec, _half_spec, _half_spec, _half_spec, _w2_spec, _b_spec],
    _half_spec,
    jax.ShapeDtypeStruct((NC, N, C0), jnp.float32),
)

_tc2_call = _tc_call(
    _tc_layer2,
    [_eps_spec, _half_spec, _half_spec, _half_spec, _w2_spec, _b_spec],
    _full_spec,
    jax.ShapeDtypeStruct((N, H), jnp.float32),
)


@jax.jit
def kernel(x, edge_index, edge_w, W1, b1, W2, b2, eps):
  src = edge_index[0]
  dst = edge_index[1]
  wbits = lax.bitcast_convert_type(edge_w, jnp.int32)
  packed = jnp.stack([src, src + N, dst, wbits], axis=1).reshape(4 * E)
  z128 = jnp.zeros((NP, C0), jnp.float32)
  b1r = b1.reshape(1, H)
  b2r = b2.reshape(1, H)

  agg0, dpart = _sc_layer0_call(x, packed, z128)
  h1 = _tc0_call(eps, x, agg0, dpart, W1, b1r)
  agg1 = _sc_layer_k_call(h1.reshape(NC * N, C0), packed, z128)
  h2 = _tc1_call(eps, h1, agg1, dpart, W2, b2r)
  agg2 = _sc_layer_k_call(h2.reshape(NC * N, C0), packed, z128)
  out = _tc2_call(eps, h2, agg2, dpart, W2, b2r)
  return out
